# baseline (speedup 1.0000x reference)
import functools
import jax
import jax.numpy as jnp
from jax import lax
from jax.experimental import pallas as pl
from jax.experimental.pallas import tpu as pltpu
from jax.experimental.pallas import tpu_sc as plsc

NC = 2
NS = 16
NW = NC * NS
CH = 128

_HI = None


def _proj_rows(et_ref, w_ref, b_ref):
    return jnp.dot(et_ref[...], w_ref[...], preferred_element_type=jnp.float32,
                   precision=_HI) + b_ref[...]


def _write_z4(o4_ref, z, t):
    for k in range(4):
        o4_ref[k] = jnp.maximum(z + t[k:k + 1, :], 0.0)


def _embed_z4_body(x_ref, tab_ref, et_ref, w_ref, b_ref, oz_ref, o4_ref):
    xi = x_ref[...]
    z = jnp.broadcast_to(tab_ref[0:1, :], oz_ref.shape)
    for k in range(1, 4):
        z = jnp.where(xi == k, tab_ref[k:k + 1, :], z)
    oz_ref[...] = z
    _write_z4(o4_ref, z, _proj_rows(et_ref, w_ref, b_ref))


def _embed_z4(x, node_table, edge_table, We_l, be_l, N, H):
    Bn = 1000
    return pl.pallas_call(
        _embed_z4_body,
        grid=(N // Bn,),
        in_specs=[
            pl.BlockSpec((Bn, 1), lambda i: (i, 0)),
            pl.BlockSpec((4, H), lambda i: (0, 0)),
            pl.BlockSpec((4, H), lambda i: (0, 0)),
            pl.BlockSpec((H, H), lambda i: (0, 0)),
            pl.BlockSpec((1, H), lambda i: (0, 0)),
        ],
        out_specs=[
            pl.BlockSpec((Bn, H), lambda i: (i, 0)),
            pl.BlockSpec((4, Bn, H), lambda i: (0, i, 0)),
        ],
        out_shape=[
            jax.ShapeDtypeStruct((N, H), jnp.float32),
            jax.ShapeDtypeStruct((4, N, H), jnp.float32),
        ],
    )(x, node_table, edge_table, We_l, be_l)


def _mlp(z_ref, a_ref, s_ref, w1_ref, b1_ref, w2_ref, b2_ref):
    a = a_ref[0] + a_ref[1]
    h = z_ref[...] * s_ref[0, 0] + a
    h = jnp.maximum(jnp.dot(h, w1_ref[...], preferred_element_type=jnp.float32,
                            precision=_HI) + b1_ref[...], 0.0)
    h = jnp.dot(h, w2_ref[...], preferred_element_type=jnp.float32,
                precision=_HI) + b2_ref[...]
    return jnp.maximum(h, 0.0)


def _upd_z4_body(z_ref, a_ref, s_ref, w1_ref, b1_ref, w2_ref, b2_ref,
                 et_ref, wn_ref, bn_ref, oz_ref, o4_ref):
    z = _mlp(z_ref, a_ref, s_ref, w1_ref, b1_ref, w2_ref, b2_ref)
    oz_ref[...] = z
    _write_z4(o4_ref, z, _proj_rows(et_ref, wn_ref, bn_ref))


def _upd_z4(z, aggr2, scale, W1_l, b1_l, W2_l, b2_l,
            edge_table, We_n, be_n, N, H):
    Bn = 1000
    return pl.pallas_call(
        _upd_z4_body,
        grid=(N // Bn,),
        in_specs=[
            pl.BlockSpec((Bn, H), lambda i: (i, 0)),
            pl.BlockSpec((2, Bn, H), lambda i: (0, i, 0)),
            pl.BlockSpec((1, 1), lambda i: (0, 0)),
            pl.BlockSpec((H, H), lambda i: (0, 0)),
            pl.BlockSpec((1, H), lambda i: (0, 0)),
            pl.BlockSpec((H, H), lambda i: (0, 0)),
            pl.BlockSpec((1, H), lambda i: (0, 0)),
            pl.BlockSpec((4, H), lambda i: (0, 0)),
            pl.BlockSpec((H, H), lambda i: (0, 0)),
            pl.BlockSpec((1, H), lambda i: (0, 0)),
        ],
        out_specs=[
            pl.BlockSpec((Bn, H), lambda i: (i, 0)),
            pl.BlockSpec((4, Bn, H), lambda i: (0, i, 0)),
        ],
        out_shape=[
            jax.ShapeDtypeStruct((N, H), jnp.float32),
            jax.ShapeDtypeStruct((4, N, H), jnp.float32),
        ],
    )(z, aggr2, scale, W1_l, b1_l, W2_l, b2_l, edge_table, We_n, be_n)


def _upd_head_body(z_ref, a_ref, s_ref, w1_ref, b1_ref, w2_ref, b2_ref,
                   wh1_ref, bh1_ref, wh2_ref, bh2_ref, o_ref):
    z = _mlp(z_ref, a_ref, s_ref, w1_ref, b1_ref, w2_ref, b2_ref)
    B = o_ref.shape[0]
    g = jnp.concatenate([z[:B], z[B:]], axis=1)
    hh = jnp.maximum(jnp.dot(g, wh1_ref[...], preferred_element_type=jnp.float32,
                             precision=_HI) + bh1_ref[...], 0.0)
    o_ref[...] = jnp.dot(hh, wh2_ref[...], preferred_element_type=jnp.float32,
                         precision=_HI) + bh2_ref[...]


def _upd_head(z, aggr2, scale, W1_l, b1_l, W2_l, b2_l,
              Wh1, bh1, Wh2, bh2, B, H):
    B2 = 2 * B
    return pl.pallas_call(
        _upd_head_body,
        grid=(1,),
        in_specs=[
            pl.BlockSpec((B2, H), lambda i: (0, 0)),
            pl.BlockSpec((2, B2, H), lambda i: (0, 0, 0)),
            pl.BlockSpec((1, 1), lambda i: (0, 0)),
            pl.BlockSpec((H, H), lambda i: (0, 0)),
            pl.BlockSpec((1, H), lambda i: (0, 0)),
            pl.BlockSpec((H, H), lambda i: (0, 0)),
            pl.BlockSpec((1, H), lambda i: (0, 0)),
            pl.BlockSpec((2 * H, H), lambda i: (0, 0)),
            pl.BlockSpec((1, H), lambda i: (0, 0)),
            pl.BlockSpec((H, 1), lambda i: (0, 0)),
            pl.BlockSpec((1, 1), lambda i: (0, 0)),
        ],
        out_specs=pl.BlockSpec((B, 1), lambda i: (0, 0)),
        out_shape=jax.ShapeDtypeStruct((B, 1), jnp.float32),
    )(z, aggr2, scale, W1_l, b1_l, W2_l, b2_l, Wh1, bh1, Wh2, bh2)


def _make_sc_aggregate(N, H, NPAD, CPW0, CPW1):
    RPS = NPAD // NS

    mesh = plsc.VectorSubcoreMesh(core_axis_name="c", subcore_axis_name="s",
                                  num_cores=NC, num_subcores=NS)

    @functools.partial(
        pl.kernel,
        out_type=jax.ShapeDtypeStruct((NC, NPAD, H), jnp.float32),
        mesh=mesh,
        scratch_types=[
            pltpu.VMEM((CH,), jnp.int32),
            pltpu.VMEM((CH,), jnp.int32),
            pltpu.VMEM((CH,), jnp.int32),
            pltpu.VMEM((CH,), jnp.int32),
            pltpu.VMEM((2, CH, H), jnp.float32),
            pltpu.VMEM((NPAD // NS // 8, H), jnp.float32),
            pltpu.VMEM_SHARED((NPAD, H), jnp.float32),
            pltpu.SemaphoreType.DMA,
            pltpu.SemaphoreType.DMA,
            pltpu.SemaphoreType.DMA,
            pltpu.SemaphoreType.DMA,
        ],
    )
    def sc_aggr(z4_hbm, gidx_hbm, dst_hbm, out_hbm,
                idxg0, idxg1, idxs0, idxs1, rows, zbuf, aggr,
                gs0, gs1, ss0, ss1, ):
        c = lax.axis_index("c")
        s = lax.axis_index("s")
        r0 = s * RPS
        cpw = jnp.where(c == 0, CPW0, CPW1)
        idxg = (idxg0, idxg1)
        idxs = (idxs0, idxs1)
        gsem = (gs0, gs1)
        ssem = (ss0, ss1)

        ZR = RPS // 8

        def zrow(i, carry):
            for k in range(H // 16):
                zbuf[i, pl.ds(k * 16, 16)] = jnp.zeros((16,), jnp.float32)
            return carry

        lax.fori_loop(0, ZR, zrow, 0)

        def zcp(r, carry):
            pltpu.sync_copy(zbuf, aggr.at[pl.ds(r0 + r * ZR, ZR)])
            return carry

        lax.fori_loop(0, 8, zcp, 0)
        plsc.subcore_barrier()

        row_base = jnp.where(c == 0, s * CPW0, NS * CPW0 + s * CPW1)

        for b in range(2):
            pltpu.sync_copy(gidx_hbm.at[row_base + b], idxg[b])
            pltpu.sync_copy(dst_hbm.at[row_base + b], idxs[b])
            pltpu.async_copy(z4_hbm.at[idxg[b]], rows.at[b], gsem[b])

        def body(jj, carry):
            for b in range(2):
                j = 2 * jj + b
                pltpu.make_async_copy(z4_hbm.at[idxg[b]], rows.at[b],
                                      gsem[b]).wait()
                sc = pltpu.async_copy(rows.at[b], aggr.at[idxs[b]], ssem[b],
                                      add=True)
                sc.wait()

                @pl.when(j + 2 < cpw)
                def _():
                    pltpu.sync_copy(gidx_hbm.at[row_base + j + 2], idxg[b])
                    pltpu.sync_copy(dst_hbm.at[row_base + j + 2], idxs[b])
                    pltpu.async_copy(z4_hbm.at[idxg[b]], rows.at[b], gsem[b])
            return carry

        lax.fori_loop(0, cpw // 2, body, 0)
        plsc.subcore_barrier()

        pltpu.sync_copy(aggr.at[pl.ds(r0, RPS)], out_hbm.at[c].at[pl.ds(r0, RPS)])

    return sc_aggr


def kernel(x, edge_type, edge_index, edge_label, node_table, edge_table, eps,
           We, be, W1, b1, W2, b2, Wh1, bh1, Wh2, bh2):
    N, H = x.shape[0], node_table.shape[1]
    E = edge_type.shape[0]
    B = edge_label.shape[0]
    L = We.shape[0]

    NPAD = ((N + NS * 8 - 1) // (NS * 8)) * (NS * 8)
    CPWT = 2 * ((E + 2 * NW * CH - 1) // (2 * NW * CH)) * 2
    CPW0 = (3 * CPWT // 4 + 1) // 2 * 2
    CPW1 = CPWT - CPW0
    E_pad = NS * CH * CPWT

    src = edge_index[0]
    dst = edge_index[1]
    gidx = (edge_type * N + src).astype(jnp.int32)
    gidx = jnp.pad(gidx, (0, E_pad - E)).reshape(E_pad // CH, CH)
    trash = N + (jnp.arange(E_pad - E, dtype=jnp.int32) % (NPAD - N))
    dstp = jnp.concatenate([dst.astype(jnp.int32), trash])
    dstp = dstp.reshape(E_pad // CH, CH)

    sc_aggr = _make_sc_aggregate(N, H, NPAD, CPW0, CPW1)

    z, z4 = _embed_z4(x, node_table, edge_table, We[0],
                      be[0].reshape(1, H), N, H)
    for l in range(L - 1):
        aggr2 = sc_aggr(z4.reshape(4 * N, H), gidx, dstp)
        scale = (1.0 + eps[l]).reshape(1, 1)
        z, z4 = _upd_z4(z, aggr2, scale, W1[l], b1[l].reshape(1, H),
                        W2[l], b2[l].reshape(1, H),
                        edge_table, We[l + 1], be[l + 1].reshape(1, H), N, H)

    aggr2 = sc_aggr(z4.reshape(4 * N, H), gidx, dstp)
    scale = (1.0 + eps[L - 1]).reshape(1, 1)
    pred = _upd_head(z[:2 * B], aggr2[:, :2 * B], scale,
                     W1[L - 1], b1[L - 1].reshape(1, H),
                     W2[L - 1], b2[L - 1].reshape(1, H),
                     Wh1, bh1.reshape(1, H), Wh2,
                     bh2.reshape(1, 1), B, H)
    return (pred, edge_label)


# speedup vs baseline: 1.0624x; 1.0624x over previous
import functools
import jax
import jax.numpy as jnp
from jax import lax
from jax.experimental import pallas as pl
from jax.experimental.pallas import tpu as pltpu
from jax.experimental.pallas import tpu_sc as plsc

NC = 2
NS = 16
NW = NC * NS
CH = 128

_HI = None


def _proj_rows(et_ref, w_ref, b_ref):
    return jnp.dot(et_ref[...], w_ref[...], preferred_element_type=jnp.float32,
                   precision=_HI) + b_ref[...]


def _write_z4(o4_ref, z, t):
    for k in range(4):
        o4_ref[k] = jnp.maximum(z + t[k:k + 1, :], 0.0)


def _embed_z4_body(x_ref, tab_ref, et_ref, w_ref, b_ref, oz_ref, o4_ref):
    xi = x_ref[...]
    z = jnp.broadcast_to(tab_ref[0:1, :], oz_ref.shape)
    for k in range(1, 4):
        z = jnp.where(xi == k, tab_ref[k:k + 1, :], z)
    oz_ref[...] = z
    _write_z4(o4_ref, z, _proj_rows(et_ref, w_ref, b_ref))


def _embed_z4(x, node_table, edge_table, We_l, be_l, N, H):
    Bn = 1000
    return pl.pallas_call(
        _embed_z4_body,
        grid=(N // Bn,),
        in_specs=[
            pl.BlockSpec((Bn, 1), lambda i: (i, 0)),
            pl.BlockSpec((4, H), lambda i: (0, 0)),
            pl.BlockSpec((4, H), lambda i: (0, 0)),
            pl.BlockSpec((H, H), lambda i: (0, 0)),
            pl.BlockSpec((1, H), lambda i: (0, 0)),
        ],
        out_specs=[
            pl.BlockSpec((Bn, H), lambda i: (i, 0)),
            pl.BlockSpec((4, Bn, H), lambda i: (0, i, 0)),
        ],
        out_shape=[
            jax.ShapeDtypeStruct((N, H), jnp.float32),
            jax.ShapeDtypeStruct((4, N, H), jnp.float32),
        ],
    )(x, node_table, edge_table, We_l, be_l)


def _mlp(z_ref, a_ref, s_ref, w1_ref, b1_ref, w2_ref, b2_ref):
    a = a_ref[0] + a_ref[1]
    h = z_ref[...] * s_ref[0, 0] + a
    h = jnp.maximum(jnp.dot(h, w1_ref[...], preferred_element_type=jnp.float32,
                            precision=_HI) + b1_ref[...], 0.0)
    h = jnp.dot(h, w2_ref[...], preferred_element_type=jnp.float32,
                precision=_HI) + b2_ref[...]
    return jnp.maximum(h, 0.0)


def _upd_z4_body(z_ref, a_ref, s_ref, w1_ref, b1_ref, w2_ref, b2_ref,
                 et_ref, wn_ref, bn_ref, oz_ref, o4_ref):
    z = _mlp(z_ref, a_ref, s_ref, w1_ref, b1_ref, w2_ref, b2_ref)
    oz_ref[...] = z
    _write_z4(o4_ref, z, _proj_rows(et_ref, wn_ref, bn_ref))


def _upd_z4(z, aggr2, scale, W1_l, b1_l, W2_l, b2_l,
            edge_table, We_n, be_n, N, H):
    Bn = 1000
    return pl.pallas_call(
        _upd_z4_body,
        grid=(N // Bn,),
        in_specs=[
            pl.BlockSpec((Bn, H), lambda i: (i, 0)),
            pl.BlockSpec((2, Bn, H), lambda i: (0, i, 0)),
            pl.BlockSpec((1, 1), lambda i: (0, 0)),
            pl.BlockSpec((H, H), lambda i: (0, 0)),
            pl.BlockSpec((1, H), lambda i: (0, 0)),
            pl.BlockSpec((H, H), lambda i: (0, 0)),
            pl.BlockSpec((1, H), lambda i: (0, 0)),
            pl.BlockSpec((4, H), lambda i: (0, 0)),
            pl.BlockSpec((H, H), lambda i: (0, 0)),
            pl.BlockSpec((1, H), lambda i: (0, 0)),
        ],
        out_specs=[
            pl.BlockSpec((Bn, H), lambda i: (i, 0)),
            pl.BlockSpec((4, Bn, H), lambda i: (0, i, 0)),
        ],
        out_shape=[
            jax.ShapeDtypeStruct((N, H), jnp.float32),
            jax.ShapeDtypeStruct((4, N, H), jnp.float32),
        ],
    )(z, aggr2, scale, W1_l, b1_l, W2_l, b2_l, edge_table, We_n, be_n)


def _upd_head_body(z_ref, a_ref, s_ref, w1_ref, b1_ref, w2_ref, b2_ref,
                   wh1_ref, bh1_ref, wh2_ref, bh2_ref, o_ref):
    z = _mlp(z_ref, a_ref, s_ref, w1_ref, b1_ref, w2_ref, b2_ref)
    B = o_ref.shape[0]
    g = jnp.concatenate([z[:B], z[B:]], axis=1)
    hh = jnp.maximum(jnp.dot(g, wh1_ref[...], preferred_element_type=jnp.float32,
                             precision=_HI) + bh1_ref[...], 0.0)
    o_ref[...] = jnp.dot(hh, wh2_ref[...], preferred_element_type=jnp.float32,
                         precision=_HI) + bh2_ref[...]


def _upd_head(z, aggr2, scale, W1_l, b1_l, W2_l, b2_l,
              Wh1, bh1, Wh2, bh2, B, H):
    B2 = 2 * B
    return pl.pallas_call(
        _upd_head_body,
        grid=(1,),
        in_specs=[
            pl.BlockSpec((B2, H), lambda i: (0, 0)),
            pl.BlockSpec((2, B2, H), lambda i: (0, 0, 0)),
            pl.BlockSpec((1, 1), lambda i: (0, 0)),
            pl.BlockSpec((H, H), lambda i: (0, 0)),
            pl.BlockSpec((1, H), lambda i: (0, 0)),
            pl.BlockSpec((H, H), lambda i: (0, 0)),
            pl.BlockSpec((1, H), lambda i: (0, 0)),
            pl.BlockSpec((2 * H, H), lambda i: (0, 0)),
            pl.BlockSpec((1, H), lambda i: (0, 0)),
            pl.BlockSpec((H, 1), lambda i: (0, 0)),
            pl.BlockSpec((1, 1), lambda i: (0, 0)),
        ],
        out_specs=pl.BlockSpec((B, 1), lambda i: (0, 0)),
        out_shape=jax.ShapeDtypeStruct((B, 1), jnp.float32),
    )(z, aggr2, scale, W1_l, b1_l, W2_l, b2_l, Wh1, bh1, Wh2, bh2)


def _make_sc_aggregate(N, H, NPAD, CPW0, CPW1):
    RPS = NPAD // NS

    mesh = plsc.VectorSubcoreMesh(core_axis_name="c", subcore_axis_name="s",
                                  num_cores=NC, num_subcores=NS)

    @functools.partial(
        pl.kernel,
        out_type=jax.ShapeDtypeStruct((NC, NPAD, H), jnp.float32),
        mesh=mesh,
        scratch_types=[
            pltpu.VMEM((CH,), jnp.int32),
            pltpu.VMEM((CH,), jnp.int32),
            pltpu.VMEM((CH,), jnp.int32),
            pltpu.VMEM((CH,), jnp.int32),
            pltpu.VMEM((2, CH, H), jnp.float32),
            pltpu.VMEM_SHARED((NPAD, H), jnp.float32),
            pltpu.SemaphoreType.DMA,
            pltpu.SemaphoreType.DMA,
            pltpu.SemaphoreType.DMA,
            pltpu.SemaphoreType.DMA,
        ],
    )
    def sc_aggr(z4_hbm, gidx_hbm, dst_hbm, zeros_hbm, out_hbm,
                idxg0, idxg1, idxs0, idxs1, rows, aggr,
                gs0, gs1, ss0, ss1, ):
        c = lax.axis_index("c")
        s = lax.axis_index("s")
        r0 = s * RPS
        cpw = jnp.where(c == 0, CPW0, CPW1)
        idxg = (idxg0, idxg1)
        idxs = (idxs0, idxs1)
        gsem = (gs0, gs1)
        ssem = (ss0, ss1)

        pltpu.sync_copy(zeros_hbm.at[pl.ds(r0, RPS)], aggr.at[pl.ds(r0, RPS)])
        plsc.subcore_barrier()

        row_base = jnp.where(c == 0, s * CPW0, NS * CPW0 + s * CPW1)

        for b in range(2):
            pltpu.sync_copy(gidx_hbm.at[row_base + b], idxg[b])
            pltpu.sync_copy(dst_hbm.at[row_base + b], idxs[b])
            pltpu.async_copy(z4_hbm.at[idxg[b]], rows.at[b], gsem[b])

        def body(jj, carry):
            for b in range(2):
                j = 2 * jj + b
                pltpu.make_async_copy(z4_hbm.at[idxg[b]], rows.at[b],
                                      gsem[b]).wait()
                sc = pltpu.async_copy(rows.at[b], aggr.at[idxs[b]], ssem[b],
                                      add=True)
                sc.wait()

                @pl.when(j + 2 < cpw)
                def _():
                    pltpu.sync_copy(gidx_hbm.at[row_base + j + 2], idxg[b])
                    pltpu.sync_copy(dst_hbm.at[row_base + j + 2], idxs[b])
                    pltpu.async_copy(z4_hbm.at[idxg[b]], rows.at[b], gsem[b])
            return carry

        lax.fori_loop(0, cpw // 2, body, 0)
        plsc.subcore_barrier()

        pltpu.sync_copy(aggr.at[pl.ds(r0, RPS)], out_hbm.at[c].at[pl.ds(r0, RPS)])

    return sc_aggr


def kernel(x, edge_type, edge_index, edge_label, node_table, edge_table, eps,
           We, be, W1, b1, W2, b2, Wh1, bh1, Wh2, bh2):
    N, H = x.shape[0], node_table.shape[1]
    E = edge_type.shape[0]
    B = edge_label.shape[0]
    L = We.shape[0]

    NPAD = ((N + NS * 8 - 1) // (NS * 8)) * (NS * 8)
    CPWT = 2 * ((E + 2 * NW * CH - 1) // (2 * NW * CH)) * 2
    CPW0 = 136
    CPW1 = CPWT - CPW0
    E_pad = NS * CH * CPWT

    src = edge_index[0]
    dst = edge_index[1]
    gidx = (edge_type * N + src).astype(jnp.int32)
    gidx = jnp.pad(gidx, (0, E_pad - E)).reshape(E_pad // CH, CH)
    trash = N + (jnp.arange(E_pad - E, dtype=jnp.int32) % (NPAD - N))
    dstp = jnp.concatenate([dst.astype(jnp.int32), trash])
    dstp = dstp.reshape(E_pad // CH, CH)

    zeros = jnp.zeros((NPAD, H), jnp.float32)
    sc_aggr = _make_sc_aggregate(N, H, NPAD, CPW0, CPW1)

    z, z4 = _embed_z4(x, node_table, edge_table, We[0],
                      be[0].reshape(1, H), N, H)
    for l in range(L - 1):
        aggr2 = sc_aggr(z4.reshape(4 * N, H), gidx, dstp, zeros)
        scale = (1.0 + eps[l]).reshape(1, 1)
        z, z4 = _upd_z4(z, aggr2, scale, W1[l], b1[l].reshape(1, H),
                        W2[l], b2[l].reshape(1, H),
                        edge_table, We[l + 1], be[l + 1].reshape(1, H), N, H)

    aggr2 = sc_aggr(z4.reshape(4 * N, H), gidx, dstp, zeros)
    scale = (1.0 + eps[L - 1]).reshape(1, 1)
    pred = _upd_head(z[:2 * B], aggr2[:, :2 * B], scale,
                     W1[L - 1], b1[L - 1].reshape(1, H),
                     W2[L - 1], b2[L - 1].reshape(1, H),
                     Wh1, bh1.reshape(1, H), Wh2,
                     bh2.reshape(1, 1), B, H)
    return (pred, edge_label)
